# baseline (speedup 1.0000x reference)
import jax
import jax.numpy as jnp
from jax import lax
from jax.experimental import pallas as pl
from jax.experimental.pallas import tpu as pltpu
from jax.experimental.pallas import tpu_sc as plsc

NUM_ENT = 1000000
NUM_REL = 1000
D = 64
B = 16384

_info = plsc.get_sparse_core_info()
NC, NS, L = _info.num_cores, _info.num_subcores, _info.num_lanes
NW = NC * NS
BW = B // NW
CH = 128
NCH = BW // CH
HALF = BW // 2


def _body(h_hbm, r_hbm, t_hbm, e_hbm, rtab_hbm, ph_hbm, pr_hbm, pt_hbm, out_hbm,
          h_v, r_v, t_v, ph_v, pr_v, pt_v, eh_v, rr_v, et_v, out_v, sem):
    wid = lax.axis_index("s") * NC + lax.axis_index("c")
    base = wid * BW

    pltpu.sync_copy(h_hbm.at[wid], h_v)
    pltpu.sync_copy(r_hbm.at[wid], r_v)
    pltpu.sync_copy(t_hbm.at[wid], t_v)
    pltpu.sync_copy(ph_hbm.at[wid], ph_v)
    pltpu.sync_copy(pr_hbm.at[wid], pr_v)
    pltpu.sync_copy(pt_hbm.at[wid], pt_v)

    lanes = lax.iota(jnp.int32, L)

    for half in range(2):
        copies = []
        for j in range(HALF // CH):
            c = half * (HALF // CH) + j
            rows = pl.ds(j * CH, CH)
            copies.append(pltpu.async_copy(e_hbm.at[h_v.at[c]], eh_v.at[rows], sem))
            copies.append(pltpu.async_copy(rtab_hbm.at[r_v.at[c]], rr_v.at[rows], sem))
            copies.append(pltpu.async_copy(e_hbm.at[t_v.at[c]], et_v.at[rows], sem))
        for cp in copies:
            cp.wait()

        def group(g, _):
            b0 = g * L
            ridx = b0 + lanes
            ch = ph_v[pl.ds(half * HALF + b0, L)]
            cr = pr_v[pl.ds(half * HALF + b0, L)]
            ct = pt_v[pl.ds(half * HALF + b0, L)]

            def dstep(d, acc):
                gh = plsc.load_gather(eh_v, [ridx, ch + d])
                gr = plsc.load_gather(rr_v, [ridx, cr + d])
                gt = plsc.load_gather(et_v, [ridx, ct + d])
                return acc + jnp.abs(gh + gr - gt)

            acc = lax.fori_loop(0, D, dstep, jnp.zeros((L,), jnp.float32))
            out_v[pl.ds(half * HALF + b0, L)] = acc
            return 0

        lax.fori_loop(0, HALF // L, group, 0)

    pltpu.sync_copy(out_v, out_hbm.at[pl.ds(base, BW)])


def kernel(h, r, t, E, R):
    h = h.astype(jnp.int32)
    r = r.astype(jnp.int32)
    t = t.astype(jnp.int32)
    E5 = E.reshape(NUM_ENT // 2, 2 * D)
    R5 = R.reshape(NUM_REL // 2, 2 * D)
    h2 = (h >> 1).reshape(NW, NCH, CH)
    r2 = (r >> 1).reshape(NW, NCH, CH)
    t2 = (t >> 1).reshape(NW, NCH, CH)
    ph = ((h & 1) * D).reshape(NW, BW)
    pr = ((r & 1) * D).reshape(NW, BW)
    pt = ((t & 1) * D).reshape(NW, BW)

    mesh = plsc.VectorSubcoreMesh(core_axis_name="c", subcore_axis_name="s")
    run = pl.kernel(
        _body,
        out_type=jax.ShapeDtypeStruct((B,), jnp.float32),
        mesh=mesh,
        compiler_params=pltpu.CompilerParams(
            needs_layout_passes=False, use_tc_tiling_on_sc=True),
        scratch_types=[
            pltpu.VMEM((NCH, CH), jnp.int32),
            pltpu.VMEM((NCH, CH), jnp.int32),
            pltpu.VMEM((NCH, CH), jnp.int32),
            pltpu.VMEM((BW,), jnp.int32),
            pltpu.VMEM((BW,), jnp.int32),
            pltpu.VMEM((BW,), jnp.int32),
            pltpu.VMEM((HALF, 2 * D), jnp.float32),
            pltpu.VMEM((HALF, 2 * D), jnp.float32),
            pltpu.VMEM((HALF, 2 * D), jnp.float32),
            pltpu.VMEM((BW,), jnp.float32),
            pltpu.SemaphoreType.DMA,
        ],
    )
    return run(h2, r2, t2, E5, R5, ph, pr, pt)
